# Initial kernel scaffold; baseline (speedup 1.0000x reference)
import jax
import jax.numpy as jnp
from jax.experimental import pallas as pl

N, D, C, H, E = 2048, 768, 32, 1536, 8


def _router_body(z_ref, rw_ref, rb_ref, probs_ref, wdense_ref):
    logits = jnp.dot(z_ref[...], rw_ref[...],
                     preferred_element_type=jnp.float32) + rb_ref[...]
    m = jnp.max(logits, axis=-1, keepdims=True)
    ex = jnp.exp(logits - m)
    probs = ex / jnp.sum(ex, axis=-1, keepdims=True)
    probs_ref[...] = probs
    idx = jax.lax.broadcasted_iota(jnp.int32, probs.shape, 1)
    m1 = jnp.max(probs, axis=-1, keepdims=True)
    i1 = jnp.min(jnp.where(probs == m1, idx, E), axis=-1, keepdims=True)
    oh1 = idx == i1
    pm = jnp.where(oh1, -1.0, probs)
    m2 = jnp.max(pm, axis=-1, keepdims=True)
    i2 = jnp.min(jnp.where(pm == m2, idx, E), axis=-1, keepdims=True)
    oh2 = idx == i2
    s = m1 + m2 + 1e-9
    wdense_ref[...] = (jnp.where(oh1, m1 / s, 0.0)
                       + jnp.where(oh2, m2 / s, 0.0))


def _moe_body(wT_ref, z_ref, cc_ref, W2_ref, b2_ref, W3_ref, W1_ref, b1_ref,
              W4_ref, b4_ref, W5_ref, b5_ref, g_ref, bn_ref, out_ref):
    e = pl.program_id(0)
    x = z_ref[...]
    cvec = jnp.dot(cc_ref[...], W3_ref[0],
                   preferred_element_type=jnp.float32) + b2_ref[...]
    h = jnp.dot(x, W2_ref[0], preferred_element_type=jnp.float32) + cvec
    h = jnp.where(h > 0, h, jnp.exp(jnp.minimum(h, 0.0)) - 1.0)
    h2 = jnp.dot(h, W1_ref[0], preferred_element_type=jnp.float32) + b1_ref[...]
    a = jnp.dot(h2, W4_ref[0], preferred_element_type=jnp.float32) + b4_ref[...]
    b = jnp.dot(h2, W5_ref[0], preferred_element_type=jnp.float32) + b5_ref[...]
    glu = a * (1.0 / (1.0 + jnp.exp(-b)))
    r = x + glu
    mu = jnp.mean(r, axis=-1, keepdims=True)
    var = jnp.mean(r * r, axis=-1, keepdims=True) - mu * mu
    y = (r - mu) * jax.lax.rsqrt(var + 1e-5) * g_ref[...] + bn_ref[...]
    y = y * wT_ref[0]

    @pl.when(e == 0)
    def _():
        out_ref[...] = jnp.zeros_like(out_ref)

    out_ref[...] += y


def kernel(z, c_c, router_W, router_b, W2, b2, W3, W1, b1, W4, b4, W5, b5,
           ln_g, ln_b):
    z2 = z.reshape(N, D)
    probs, wdense = pl.pallas_call(
        _router_body,
        out_shape=(
            jax.ShapeDtypeStruct((N, E), jnp.float32),
            jax.ShapeDtypeStruct((N, E), jnp.float32),
        ),
    )(z2, router_W, router_b.reshape(1, E))

    wT = wdense.T.reshape(E, N, 1)

    full = lambda i: (0, 0)

    out = pl.pallas_call(
        _moe_body,
        grid=(E,),
        in_specs=[
            pl.BlockSpec((1, N, 1), lambda i: (i, 0, 0)),
            pl.BlockSpec((N, D), full),
            pl.BlockSpec((1, C), full),
            pl.BlockSpec((1, D, H), lambda i: (i, 0, 0)),
            pl.BlockSpec((1, H), lambda i: (i, 0)),
            pl.BlockSpec((1, C, H), lambda i: (i, 0, 0)),
            pl.BlockSpec((1, H, D), lambda i: (i, 0, 0)),
            pl.BlockSpec((1, D), lambda i: (i, 0)),
            pl.BlockSpec((1, D, D), lambda i: (i, 0, 0)),
            pl.BlockSpec((1, D), lambda i: (i, 0)),
            pl.BlockSpec((1, D, D), lambda i: (i, 0, 0)),
            pl.BlockSpec((1, D), lambda i: (i, 0)),
            pl.BlockSpec((1, D), lambda i: (i, 0)),
            pl.BlockSpec((1, D), lambda i: (i, 0)),
        ],
        out_specs=pl.BlockSpec((N, D), full),
        out_shape=jax.ShapeDtypeStruct((N, D), jnp.float32),
    )(wT, z2, c_c, W2, b2, W3, W1, b1, W4, b4, W5, b5, ln_g, ln_b)

    return out.reshape(1, N, D), probs.reshape(1, N, E)


# speedup vs baseline: 2.2547x; 2.2547x over previous
import jax
import jax.numpy as jnp
from jax.experimental import pallas as pl
from jax.experimental.pallas import tpu as pltpu

N, D, C, H, E = 2048, 768, 32, 1536, 8
TB = 512
NT = N // TB


def _router_body(z_ref, rw_ref, rb_ref, probs_ref, wdense_ref):
    logits = jnp.dot(z_ref[...], rw_ref[...],
                     preferred_element_type=jnp.float32) + rb_ref[...]
    m = jnp.max(logits, axis=-1, keepdims=True)
    ex = jnp.exp(logits - m)
    probs = ex / jnp.sum(ex, axis=-1, keepdims=True)
    probs_ref[...] = probs
    idx = jax.lax.broadcasted_iota(jnp.int32, probs.shape, 1)
    m1 = jnp.max(probs, axis=-1, keepdims=True)
    i1 = jnp.min(jnp.where(probs == m1, idx, E), axis=-1, keepdims=True)
    oh1 = idx == i1
    pm = jnp.where(oh1, -1.0, probs)
    m2 = jnp.max(pm, axis=-1, keepdims=True)
    i2 = jnp.min(jnp.where(pm == m2, idx, E), axis=-1, keepdims=True)
    oh2 = idx == i2
    s = m1 + m2 + 1e-9
    wdense_ref[...] = (jnp.where(oh1, m1 / s, 0.0)
                       + jnp.where(oh2, m2 / s, 0.0))


def _moe_body(wT_ref, z_ref, cc_ref, W2_ref, b2_ref, W3_ref, W1_ref, b1_ref,
              W4_ref, b4_ref, W5_ref, b5_ref, g_ref, bn_ref, out_ref,
              acc_ref):
    e = pl.program_id(0)
    j = pl.program_id(1)
    x = z_ref[...]
    cvec = jnp.dot(cc_ref[...], W3_ref[0],
                   preferred_element_type=jnp.float32) + b2_ref[0]
    h = jnp.dot(x, W2_ref[0], preferred_element_type=jnp.float32) + cvec
    h = jnp.where(h > 0, h, jnp.exp(jnp.minimum(h, 0.0)) - 1.0)
    h2 = jnp.dot(h, W1_ref[0], preferred_element_type=jnp.float32) + b1_ref[0]
    a = jnp.dot(h2, W4_ref[0], preferred_element_type=jnp.float32) + b4_ref[0]
    b = jnp.dot(h2, W5_ref[0], preferred_element_type=jnp.float32) + b5_ref[0]
    glu = a * (1.0 / (1.0 + jnp.exp(-b)))
    r = x + glu
    mu = jnp.mean(r, axis=-1, keepdims=True)
    var = jnp.mean(r * r, axis=-1, keepdims=True) - mu * mu
    y = (r - mu) * jax.lax.rsqrt(var + 1e-5) * g_ref[0] + bn_ref[0]
    y = y * wT_ref[0]

    sl = pl.ds(j * TB, TB)

    @pl.when(e == 0)
    def _():
        acc_ref[sl, :] = jnp.zeros_like(y)

    acc_ref[sl, :] += y
    out_ref[...] = acc_ref[sl, :]


def kernel(z, c_c, router_W, router_b, W2, b2, W3, W1, b1, W4, b4, W5, b5,
           ln_g, ln_b):
    z2 = z.reshape(N, D)
    probs, wdense = pl.pallas_call(
        _router_body,
        out_shape=(
            jax.ShapeDtypeStruct((N, E), jnp.float32),
            jax.ShapeDtypeStruct((N, E), jnp.float32),
        ),
    )(z2, router_W, router_b.reshape(1, E))

    wT = wdense.T.reshape(E, N, 1)

    ee = lambda e, j: (e, 0, 0)

    out = pl.pallas_call(
        _moe_body,
        grid=(E, NT),
        in_specs=[
            pl.BlockSpec((1, TB, 1), lambda e, j: (e, j, 0)),
            pl.BlockSpec((TB, D), lambda e, j: (j, 0)),
            pl.BlockSpec((1, C), lambda e, j: (0, 0)),
            pl.BlockSpec((1, D, H), ee),
            pl.BlockSpec((1, 1, H), ee),
            pl.BlockSpec((1, C, H), ee),
            pl.BlockSpec((1, H, D), ee),
            pl.BlockSpec((1, 1, D), ee),
            pl.BlockSpec((1, D, D), ee),
            pl.BlockSpec((1, 1, D), ee),
            pl.BlockSpec((1, D, D), ee),
            pl.BlockSpec((1, 1, D), ee),
            pl.BlockSpec((1, 1, D), ee),
            pl.BlockSpec((1, 1, D), ee),
        ],
        out_specs=pl.BlockSpec((TB, D), lambda e, j: (j, 0)),
        out_shape=jax.ShapeDtypeStruct((N, D), jnp.float32),
        scratch_shapes=[pltpu.VMEM((N, D), jnp.float32)],
    )(wT, z2, c_c, W2, b2.reshape(E, 1, H), W3, W1, b1.reshape(E, 1, D),
      W4, b4.reshape(E, 1, D), W5, b5.reshape(E, 1, D),
      ln_g.reshape(E, 1, D), ln_b.reshape(E, 1, D))

    return out.reshape(1, N, D), probs.reshape(1, N, E)


# speedup vs baseline: 3.0853x; 1.3684x over previous
import functools

import jax
import jax.numpy as jnp
from jax import lax
from jax.experimental import pallas as pl
from jax.experimental.pallas import tpu as pltpu
from jax.experimental.pallas import tpu_sc as plsc

N, D, C, H, E = 2048, 768, 32, 1536, 8
MT = 256
T = 24
P = MT * T
NW = 32
TOK = N // NW
L = 16


def _plan_body(z_ref, rw_ref, rb_ref, probs_ref, dA_ref, dB_ref,
               wA_ref, wB_ref, te_ref, tv_ref):
    logits = jnp.dot(z_ref[...], rw_ref[...],
                     preferred_element_type=jnp.float32) + rb_ref[...]
    m = jnp.max(logits, axis=-1, keepdims=True)
    ex = jnp.exp(logits - m)
    probs = ex / jnp.sum(ex, axis=-1, keepdims=True)
    probs_ref[...] = probs

    idx = lax.broadcasted_iota(jnp.int32, (N, E), 1)
    m1 = jnp.max(probs, axis=-1, keepdims=True)
    i1 = jnp.min(jnp.where(probs == m1, idx, E), axis=-1, keepdims=True)
    ohA = (idx == i1)
    pm = jnp.where(ohA, -1.0, probs)
    m2 = jnp.max(pm, axis=-1, keepdims=True)
    i2 = jnp.min(jnp.where(pm == m2, idx, E), axis=-1, keepdims=True)
    ohB = (idx == i2)
    s = m1 + m2 + 1e-9
    wA_ref[...] = jnp.broadcast_to(m1 / s, (N, L))
    wB_ref[...] = jnp.broadcast_to(m2 / s, (N, L))

    ohAf = ohA.astype(jnp.float32)
    ohBf = ohB.astype(jnp.float32)
    bs = 128
    li = (lax.broadcasted_iota(jnp.int32, (bs, bs), 0)
          > lax.broadcasted_iota(jnp.int32, (bs, bs), 1)).astype(jnp.float32)
    carry = jnp.zeros((1, E), jnp.float32)
    parts = {0: [], 1: []}
    for half, oh in ((0, ohAf), (1, ohBf)):
        for blk in range(N // bs):
            xb = oh[blk * bs:(blk + 1) * bs]
            excl = jnp.dot(li, xb, preferred_element_type=jnp.float32) + carry
            parts[half].append(jnp.sum(excl * xb, axis=1, keepdims=True))
            carry = carry + jnp.sum(xb, axis=0, keepdims=True)
    rankA = jnp.concatenate(parts[0], axis=0)
    rankB = jnp.concatenate(parts[1], axis=0)
    counts = carry.astype(jnp.int32)
    padded = ((counts + (MT - 1)) // MT) * MT
    ut8 = (lax.broadcasted_iota(jnp.int32, (E, E), 0)
           <= lax.broadcasted_iota(jnp.int32, (E, E), 1)).astype(jnp.float32)
    offs_incl = jnp.dot(padded.astype(jnp.float32), ut8,
                        preferred_element_type=jnp.float32).astype(jnp.int32)
    offs_excl = offs_incl - padded
    offs_excl_f = offs_excl.astype(jnp.float32)
    dA_ref[...] = (jnp.sum(ohAf * offs_excl_f, axis=1, keepdims=True)
                   + rankA).astype(jnp.int32)
    dB_ref[...] = (jnp.sum(ohBf * offs_excl_f, axis=1, keepdims=True)
                   + rankB).astype(jnp.int32)

    jv = lax.broadcasted_iota(jnp.int32, (1, T), 1)
    ot = offs_excl // MT
    acc = jnp.full((1, T), -1, jnp.int32)
    for e in range(E):
        acc = acc + (jv >= ot[0:1, e:e + 1]).astype(jnp.int32)
    te_ref[...] = acc
    tv_ref[...] = (jv < offs_incl[0:1, E - 1:E] // MT).astype(jnp.int32)


def _run_plan(z2, router_W, router_b):
    return pl.pallas_call(
        _plan_body,
        out_shape=(
            jax.ShapeDtypeStruct((N, E), jnp.float32),
            jax.ShapeDtypeStruct((N, 1), jnp.int32),
            jax.ShapeDtypeStruct((N, 1), jnp.int32),
            jax.ShapeDtypeStruct((N, L), jnp.float32),
            jax.ShapeDtypeStruct((N, L), jnp.float32),
            jax.ShapeDtypeStruct((1, T), jnp.int32),
            jax.ShapeDtypeStruct((1, T), jnp.int32),
        ),
    )(z2, router_W, router_b.reshape(1, E))


@functools.cache
def _make_sc_dispatch():
    mesh = plsc.VectorSubcoreMesh(core_axis_name="c", subcore_axis_name="s")

    @functools.partial(
        pl.kernel,
        out_type=jax.ShapeDtypeStruct((P, D), jnp.float32),
        mesh=mesh,
        scratch_types=[
            pltpu.VMEM((TOK, D), jnp.float32),
            pltpu.VMEM((TOK,), jnp.int32),
            pltpu.VMEM((TOK,), jnp.int32),
            pltpu.SemaphoreType.DMA,
        ],
    )
    def _sc_dispatch_k(z_hbm, dA_hbm, dB_hbm, xs_hbm, rows_v, idxA_v,
                       idxB_v, sem):
        wid = lax.axis_index("s") * 2 + lax.axis_index("c")
        base = wid * TOK
        pltpu.sync_copy(z_hbm.at[pl.ds(base, TOK)], rows_v)
        pltpu.sync_copy(dA_hbm.at[pl.ds(base, TOK)], idxA_v)
        pltpu.sync_copy(dB_hbm.at[pl.ds(base, TOK)], idxB_v)
        pltpu.async_copy(rows_v, xs_hbm.at[idxA_v], sem).wait()
        pltpu.async_copy(rows_v, xs_hbm.at[idxB_v], sem).wait()

    return _sc_dispatch_k


def _sc_dispatch(z2, dA1, dB1):
    return _make_sc_dispatch()(z2, dA1, dB1)


def _grn_body(te_ref, tv_ref, x_ref, cc_ref, W2_ref, b2_ref, W3_ref, W1_ref,
              b1_ref, W4_ref, b4_ref, W5_ref, b5_ref, g_ref, bn_ref, y_ref):
    j = pl.program_id(0)

    @pl.when(tv_ref[j] == 1)
    def _():
        x = x_ref[...]
        cvec = jnp.dot(cc_ref[...], W3_ref[0],
                       preferred_element_type=jnp.float32) + b2_ref[0]
        h = jnp.dot(x, W2_ref[0], preferred_element_type=jnp.float32) + cvec
        h = jnp.where(h > 0, h, jnp.exp(jnp.minimum(h, 0.0)) - 1.0)
        h2 = jnp.dot(h, W1_ref[0],
                     preferred_element_type=jnp.float32) + b1_ref[0]
        a = jnp.dot(h2, W4_ref[0],
                    preferred_element_type=jnp.float32) + b4_ref[0]
        b = jnp.dot(h2, W5_ref[0],
                    preferred_element_type=jnp.float32) + b5_ref[0]
        glu = a * (1.0 / (1.0 + jnp.exp(-b)))
        r = x + glu
        mu = jnp.mean(r, axis=-1, keepdims=True)
        var = jnp.mean(r * r, axis=-1, keepdims=True) - mu * mu
        y_ref[...] = ((r - mu) * lax.rsqrt(var + 1e-5) * g_ref[0] + bn_ref[0])


def _run_grn(te, tv, xs, c_c, W2, b2, W3, W1, b1, W4, b4, W5, b5, ln_g, ln_b):
    ee = lambda j, te_r, tv_r: (te_r[j], 0, 0)
    grid_spec = pltpu.PrefetchScalarGridSpec(
        num_scalar_prefetch=2,
        grid=(T,),
        in_specs=[
            pl.BlockSpec((MT, D), lambda j, te_r, tv_r: (j, 0)),
            pl.BlockSpec((1, C), lambda j, te_r, tv_r: (0, 0)),
            pl.BlockSpec((1, D, H), ee),
            pl.BlockSpec((1, 1, H), ee),
            pl.BlockSpec((1, C, H), ee),
            pl.BlockSpec((1, H, D), ee),
            pl.BlockSpec((1, 1, D), ee),
            pl.BlockSpec((1, D, D), ee),
            pl.BlockSpec((1, 1, D), ee),
            pl.BlockSpec((1, D, D), ee),
            pl.BlockSpec((1, 1, D), ee),
            pl.BlockSpec((1, 1, D), ee),
            pl.BlockSpec((1, 1, D), ee),
        ],
        out_specs=pl.BlockSpec((MT, D), lambda j, te_r, tv_r: (j, 0)),
    )
    return pl.pallas_call(
        _grn_body,
        grid_spec=grid_spec,
        out_shape=jax.ShapeDtypeStruct((P, D), jnp.float32),
    )(te, tv, xs, c_c, W2, b2.reshape(E, 1, H), W3, W1, b1.reshape(E, 1, D),
      W4, b4.reshape(E, 1, D), W5, b5.reshape(E, 1, D),
      ln_g.reshape(E, 1, D), ln_b.reshape(E, 1, D))


@functools.cache
def _make_sc_combine():
    mesh = plsc.VectorSubcoreMesh(core_axis_name="c", subcore_axis_name="s")

    @functools.partial(
        pl.kernel,
        out_type=jax.ShapeDtypeStruct((N, D), jnp.float32),
        mesh=mesh,
        scratch_types=[
            pltpu.VMEM((TOK, D), jnp.float32),
            pltpu.VMEM((TOK, D), jnp.float32),
            pltpu.VMEM((TOK,), jnp.int32),
            pltpu.VMEM((TOK,), jnp.int32),
            pltpu.VMEM((TOK, L), jnp.float32),
            pltpu.VMEM((TOK, L), jnp.float32),
            pltpu.SemaphoreType.DMA,
        ],
    )
    def _sc_combine_k(y_hbm, dA_hbm, dB_hbm, wA_hbm, wB_hbm, out_hbm,
                      bufA, bufB, idxA, idxB, wAv, wBv, sem):
        wid = lax.axis_index("s") * 2 + lax.axis_index("c")
        base = wid * TOK
        pltpu.sync_copy(dA_hbm.at[pl.ds(base, TOK)], idxA)
        pltpu.sync_copy(dB_hbm.at[pl.ds(base, TOK)], idxB)
        pltpu.sync_copy(wA_hbm.at[pl.ds(base, TOK)], wAv)
        pltpu.sync_copy(wB_hbm.at[pl.ds(base, TOK)], wBv)
        pltpu.async_copy(y_hbm.at[idxA], bufA, sem).wait()
        pltpu.async_copy(y_hbm.at[idxB], bufB, sem).wait()

        def row(r, carry):
            wa = wAv[r, :]
            wb = wBv[r, :]

            def chunk(k, c2):
                sl = pl.ds(k * L, L)
                bufA[r, sl] = bufA[r, sl] * wa + bufB[r, sl] * wb
                return c2

            lax.fori_loop(0, D // L, chunk, 0)
            return carry

        lax.fori_loop(0, TOK, row, 0)
        pltpu.sync_copy(bufA, out_hbm.at[pl.ds(base, TOK)])

    return _sc_combine_k


def _sc_combine(ys, dA1, dB1, wA, wB):
    return _make_sc_combine()(ys, dA1, dB1, wA, wB)


def kernel(z, c_c, router_W, router_b, W2, b2, W3, W1, b1, W4, b4, W5, b5,
           ln_g, ln_b):
    z2 = z.reshape(N, D)
    probs, dA, dB, wA, wB, te, tv = _run_plan(z2, router_W, router_b)
    dA1 = dA.reshape(N)
    dB1 = dB.reshape(N)
    xs = _sc_dispatch(z2, dA1, dB1)
    ys = _run_grn(te.reshape(T), tv.reshape(T), xs, c_c, W2, b2, W3, W1, b1,
                  W4, b4, W5, b5, ln_g, ln_b)
    out = _sc_combine(ys, dA1, dB1, wA, wB)
    return out.reshape(1, N, D), probs.reshape(1, N, E)


# speedup vs baseline: 3.3739x; 1.0935x over previous
import functools

import jax
import jax.numpy as jnp
from jax import lax
from jax.experimental import pallas as pl
from jax.experimental.pallas import tpu as pltpu
from jax.experimental.pallas import tpu_sc as plsc

N, D, C, H, E = 2048, 768, 32, 1536, 8
MT = 256
T = 24
P = MT * T
NW = 32
TOK = N // NW
L = 16


def _plan_body(z_ref, rw_ref, rb_ref, probs_ref, dA_ref, dB_ref,
               wA_ref, wB_ref, te_ref, tv_ref):
    logits = jnp.dot(z_ref[...], rw_ref[...],
                     preferred_element_type=jnp.float32) + rb_ref[...]
    m = jnp.max(logits, axis=-1, keepdims=True)
    ex = jnp.exp(logits - m)
    probs = ex / jnp.sum(ex, axis=-1, keepdims=True)
    probs_ref[...] = probs

    idx = lax.broadcasted_iota(jnp.int32, (N, E), 1)
    m1 = jnp.max(probs, axis=-1, keepdims=True)
    i1 = jnp.min(jnp.where(probs == m1, idx, E), axis=-1, keepdims=True)
    ohA = (idx == i1)
    pm = jnp.where(ohA, -1.0, probs)
    m2 = jnp.max(pm, axis=-1, keepdims=True)
    i2 = jnp.min(jnp.where(pm == m2, idx, E), axis=-1, keepdims=True)
    ohB = (idx == i2)
    s = m1 + m2 + 1e-9
    wA_ref[...] = jnp.broadcast_to(m1 / s, (N, L))
    wB_ref[...] = jnp.broadcast_to(m2 / s, (N, L))

    ohAf = ohA.astype(jnp.float32)
    ohBf = ohB.astype(jnp.float32)
    bs = 128
    li = (lax.broadcasted_iota(jnp.int32, (bs, bs), 0)
          > lax.broadcasted_iota(jnp.int32, (bs, bs), 1)).astype(jnp.float32)
    carry = jnp.zeros((1, E), jnp.float32)
    parts = {0: [], 1: []}
    for half, oh in ((0, ohAf), (1, ohBf)):
        for blk in range(N // bs):
            xb = oh[blk * bs:(blk + 1) * bs]
            excl = jnp.dot(li, xb, preferred_element_type=jnp.float32) + carry
            parts[half].append(jnp.sum(excl * xb, axis=1, keepdims=True))
            carry = carry + jnp.sum(xb, axis=0, keepdims=True)
    rankA = jnp.concatenate(parts[0], axis=0)
    rankB = jnp.concatenate(parts[1], axis=0)
    counts = carry.astype(jnp.int32)
    padded = ((counts + (MT - 1)) // MT) * MT
    ut8 = (lax.broadcasted_iota(jnp.int32, (E, E), 0)
           <= lax.broadcasted_iota(jnp.int32, (E, E), 1)).astype(jnp.float32)
    offs_incl = jnp.dot(padded.astype(jnp.float32), ut8,
                        preferred_element_type=jnp.float32).astype(jnp.int32)
    offs_excl = offs_incl - padded
    offs_excl_f = offs_excl.astype(jnp.float32)
    dA_ref[...] = (jnp.sum(ohAf * offs_excl_f, axis=1, keepdims=True)
                   + rankA).astype(jnp.int32)
    dB_ref[...] = (jnp.sum(ohBf * offs_excl_f, axis=1, keepdims=True)
                   + rankB).astype(jnp.int32)

    jv = lax.broadcasted_iota(jnp.int32, (1, T), 1)
    ot = offs_excl // MT
    acc = jnp.full((1, T), -1, jnp.int32)
    for e in range(E):
        acc = acc + (jv >= ot[0:1, e:e + 1]).astype(jnp.int32)
    te_ref[...] = acc
    tv_ref[...] = (jv < offs_incl[0:1, E - 1:E] // MT).astype(jnp.int32)


def _run_plan(z2, router_W, router_b):
    return pl.pallas_call(
        _plan_body,
        out_shape=(
            jax.ShapeDtypeStruct((N, E), jnp.float32),
            jax.ShapeDtypeStruct((N, 1), jnp.int32),
            jax.ShapeDtypeStruct((N, 1), jnp.int32),
            jax.ShapeDtypeStruct((N, L), jnp.float32),
            jax.ShapeDtypeStruct((N, L), jnp.float32),
            jax.ShapeDtypeStruct((1, T), jnp.int32),
            jax.ShapeDtypeStruct((1, T), jnp.int32),
        ),
    )(z2, router_W, router_b.reshape(1, E))


@functools.cache
def _make_sc_dispatch():
    mesh = plsc.VectorSubcoreMesh(core_axis_name="c", subcore_axis_name="s")

    @functools.partial(
        pl.kernel,
        out_type=jax.ShapeDtypeStruct((P, D), jnp.float32),
        mesh=mesh,
        scratch_types=[
            pltpu.VMEM((TOK, D), jnp.float32),
            pltpu.VMEM((TOK,), jnp.int32),
            pltpu.VMEM((TOK,), jnp.int32),
            pltpu.SemaphoreType.DMA,
        ],
    )
    def _sc_dispatch_k(z_hbm, dA_hbm, dB_hbm, xs_hbm, rows_v, idxA_v,
                       idxB_v, sem):
        wid = lax.axis_index("s") * 2 + lax.axis_index("c")
        base = wid * TOK
        pltpu.sync_copy(z_hbm.at[pl.ds(base, TOK)], rows_v)
        pltpu.sync_copy(dA_hbm.at[pl.ds(base, TOK)], idxA_v)
        pltpu.sync_copy(dB_hbm.at[pl.ds(base, TOK)], idxB_v)
        pltpu.async_copy(rows_v, xs_hbm.at[idxA_v], sem).wait()
        pltpu.async_copy(rows_v, xs_hbm.at[idxB_v], sem).wait()

    return _sc_dispatch_k


def _sc_dispatch(z2, dA1, dB1):
    return _make_sc_dispatch()(z2, dA1, dB1)


def _grn_body(te_ref, tv_ref, x_ref, cc_ref, W2_ref, b2_ref, W3_ref, W1_ref,
              b1_ref, W4_ref, b4_ref, W5_ref, b5_ref, g_ref, bn_ref, y_ref):
    j = pl.program_id(0)

    @pl.when(tv_ref[j] == 1)
    def _():
        x = x_ref[...]
        cvec = jnp.dot(cc_ref[...], W3_ref[0],
                       preferred_element_type=jnp.float32) + b2_ref[0]
        h = jnp.dot(x, W2_ref[0], preferred_element_type=jnp.float32) + cvec
        h = jnp.where(h > 0, h, jnp.exp(jnp.minimum(h, 0.0)) - 1.0)
        h2 = jnp.dot(h, W1_ref[0],
                     preferred_element_type=jnp.float32) + b1_ref[0]
        a = jnp.dot(h2, W4_ref[0],
                    preferred_element_type=jnp.float32) + b4_ref[0]
        b = jnp.dot(h2, W5_ref[0],
                    preferred_element_type=jnp.float32) + b5_ref[0]
        glu = a * (1.0 / (1.0 + jnp.exp(-b)))
        r = x + glu
        mu = jnp.mean(r, axis=-1, keepdims=True)
        var = jnp.mean(r * r, axis=-1, keepdims=True) - mu * mu
        y_ref[...] = ((r - mu) * lax.rsqrt(var + 1e-5) * g_ref[0] + bn_ref[0])


def _run_grn(te, tv, xs, c_c, W2, b2, W3, W1, b1, W4, b4, W5, b5, ln_g, ln_b):
    ee = lambda j, te_r, tv_r: (te_r[j], 0, 0)
    grid_spec = pltpu.PrefetchScalarGridSpec(
        num_scalar_prefetch=2,
        grid=(T,),
        in_specs=[
            pl.BlockSpec((MT, D), lambda j, te_r, tv_r: (j, 0)),
            pl.BlockSpec((1, C), lambda j, te_r, tv_r: (0, 0)),
            pl.BlockSpec((1, D, H), ee),
            pl.BlockSpec((1, 1, H), ee),
            pl.BlockSpec((1, C, H), ee),
            pl.BlockSpec((1, H, D), ee),
            pl.BlockSpec((1, 1, D), ee),
            pl.BlockSpec((1, D, D), ee),
            pl.BlockSpec((1, 1, D), ee),
            pl.BlockSpec((1, D, D), ee),
            pl.BlockSpec((1, 1, D), ee),
            pl.BlockSpec((1, 1, D), ee),
            pl.BlockSpec((1, 1, D), ee),
        ],
        out_specs=pl.BlockSpec((MT, D), lambda j, te_r, tv_r: (j, 0)),
    )
    return pl.pallas_call(
        _grn_body,
        grid_spec=grid_spec,
        out_shape=jax.ShapeDtypeStruct((P, D), jnp.float32),
    )(te, tv, xs, c_c, W2, b2.reshape(E, 1, H), W3, W1, b1.reshape(E, 1, D),
      W4, b4.reshape(E, 1, D), W5, b5.reshape(E, 1, D),
      ln_g.reshape(E, 1, D), ln_b.reshape(E, 1, D))


@functools.cache
def _make_sc_combine():
    mesh = plsc.VectorSubcoreMesh(core_axis_name="c", subcore_axis_name="s")

    @functools.partial(
        pl.kernel,
        out_type=jax.ShapeDtypeStruct((N, D), jnp.float32),
        mesh=mesh,
        scratch_types=[
            pltpu.VMEM((TOK, D), jnp.float32),
            pltpu.VMEM((TOK, D), jnp.float32),
            pltpu.VMEM((TOK,), jnp.int32),
            pltpu.VMEM((TOK,), jnp.int32),
            pltpu.VMEM((TOK, L), jnp.float32),
            pltpu.VMEM((TOK, L), jnp.float32),
            pltpu.SemaphoreType.DMA,
        ],
    )
    def _sc_combine_k(y_hbm, dA_hbm, dB_hbm, wA_hbm, wB_hbm, out_hbm,
                      bufA, bufB, idxA, idxB, wAv, wBv, sem):
        wid = lax.axis_index("s") * 2 + lax.axis_index("c")
        base = wid * TOK
        pltpu.sync_copy(dA_hbm.at[pl.ds(base, TOK)], idxA)
        pltpu.sync_copy(dB_hbm.at[pl.ds(base, TOK)], idxB)
        pltpu.sync_copy(wA_hbm.at[pl.ds(base, TOK)], wAv)
        pltpu.sync_copy(wB_hbm.at[pl.ds(base, TOK)], wBv)
        pltpu.async_copy(y_hbm.at[idxA], bufA, sem).wait()
        pltpu.async_copy(y_hbm.at[idxB], bufB, sem).wait()

        def row(r, carry):
            wa = wAv[r, :]
            wb = wBv[r, :]
            for k in range(D // L):
                sl = pl.ds(k * L, L)
                bufA[r, sl] = bufA[r, sl] * wa + bufB[r, sl] * wb
            return carry

        lax.fori_loop(0, TOK, row, 0)
        pltpu.sync_copy(bufA, out_hbm.at[pl.ds(base, TOK)])

    return _sc_combine_k


def _sc_combine(ys, dA1, dB1, wA, wB):
    return _make_sc_combine()(ys, dA1, dB1, wA, wB)


def kernel(z, c_c, router_W, router_b, W2, b2, W3, W1, b1, W4, b4, W5, b5,
           ln_g, ln_b):
    z2 = z.reshape(N, D)
    probs, dA, dB, wA, wB, te, tv = _run_plan(z2, router_W, router_b)
    dA1 = dA.reshape(N)
    dB1 = dB.reshape(N)
    xs = _sc_dispatch(z2, dA1, dB1)
    ys = _run_grn(te.reshape(T), tv.reshape(T), xs, c_c, W2, b2, W3, W1, b1,
                  W4, b4, W5, b5, ln_g, ln_b)
    out = _sc_combine(ys, dA1, dB1, wA, wB)
    return out.reshape(1, N, D), probs.reshape(1, N, E)


# speedup vs baseline: 3.3809x; 1.0021x over previous
import functools

import jax
import jax.numpy as jnp
from jax import lax
from jax.experimental import pallas as pl
from jax.experimental.pallas import tpu as pltpu
from jax.experimental.pallas import tpu_sc as plsc

N, D, C, H, E = 2048, 768, 32, 1536, 8
MT = 256
T = 24
P = MT * T
NW = 32
TOK = N // NW
L = 16
WL = 128


def _plan_body(z_ref, rw_ref, rb_ref, probs_ref, dA_ref, dB_ref,
               wA_ref, wB_ref, te_ref, tv_ref):
    logits = jnp.dot(z_ref[...], rw_ref[...],
                     preferred_element_type=jnp.float32) + rb_ref[...]
    m = jnp.max(logits, axis=-1, keepdims=True)
    ex = jnp.exp(logits - m)
    probs = ex / jnp.sum(ex, axis=-1, keepdims=True)
    probs_ref[...] = probs

    idx = lax.broadcasted_iota(jnp.int32, (N, E), 1)
    m1 = jnp.max(probs, axis=-1, keepdims=True)
    i1 = jnp.min(jnp.where(probs == m1, idx, E), axis=-1, keepdims=True)
    ohA = (idx == i1)
    pm = jnp.where(ohA, -1.0, probs)
    m2 = jnp.max(pm, axis=-1, keepdims=True)
    i2 = jnp.min(jnp.where(pm == m2, idx, E), axis=-1, keepdims=True)
    ohB = (idx == i2)
    s = m1 + m2 + 1e-9
    wA_ref[...] = jnp.broadcast_to(m1 / s, (N, WL))
    wB_ref[...] = jnp.broadcast_to(m2 / s, (N, WL))

    ohAf = ohA.astype(jnp.float32)
    ohBf = ohB.astype(jnp.float32)
    bs = 128
    li = (lax.broadcasted_iota(jnp.int32, (bs, bs), 0)
          > lax.broadcasted_iota(jnp.int32, (bs, bs), 1)).astype(jnp.float32)
    carry = jnp.zeros((1, E), jnp.float32)
    parts = {0: [], 1: []}
    for half, oh in ((0, ohAf), (1, ohBf)):
        for blk in range(N // bs):
            xb = oh[blk * bs:(blk + 1) * bs]
            excl = jnp.dot(li, xb, preferred_element_type=jnp.float32) + carry
            parts[half].append(jnp.sum(excl * xb, axis=1, keepdims=True))
            carry = carry + jnp.sum(xb, axis=0, keepdims=True)
    rankA = jnp.concatenate(parts[0], axis=0)
    rankB = jnp.concatenate(parts[1], axis=0)
    counts = carry.astype(jnp.int32)
    padded = ((counts + (MT - 1)) // MT) * MT
    ut8 = (lax.broadcasted_iota(jnp.int32, (E, E), 0)
           <= lax.broadcasted_iota(jnp.int32, (E, E), 1)).astype(jnp.float32)
    offs_incl = jnp.dot(padded.astype(jnp.float32), ut8,
                        preferred_element_type=jnp.float32).astype(jnp.int32)
    offs_excl = offs_incl - padded
    offs_excl_f = offs_excl.astype(jnp.float32)
    dA_ref[...] = (jnp.sum(ohAf * offs_excl_f, axis=1, keepdims=True)
                   + rankA).astype(jnp.int32)
    dB_ref[...] = (jnp.sum(ohBf * offs_excl_f, axis=1, keepdims=True)
                   + rankB).astype(jnp.int32)

    jv = lax.broadcasted_iota(jnp.int32, (1, T), 1)
    ot = offs_excl // MT
    acc = jnp.full((1, T), -1, jnp.int32)
    for e in range(E):
        acc = acc + (jv >= ot[0:1, e:e + 1]).astype(jnp.int32)
    te_ref[...] = acc
    tv_ref[...] = (jv < offs_incl[0:1, E - 1:E] // MT).astype(jnp.int32)


def _run_plan(z2, router_W, router_b):
    return pl.pallas_call(
        _plan_body,
        out_shape=(
            jax.ShapeDtypeStruct((N, E), jnp.float32),
            jax.ShapeDtypeStruct((N, 1), jnp.int32),
            jax.ShapeDtypeStruct((N, 1), jnp.int32),
            jax.ShapeDtypeStruct((N, WL), jnp.float32),
            jax.ShapeDtypeStruct((N, WL), jnp.float32),
            jax.ShapeDtypeStruct((1, T), jnp.int32),
            jax.ShapeDtypeStruct((1, T), jnp.int32),
        ),
    )(z2, router_W, router_b.reshape(1, E))


@functools.cache
def _make_sc_dispatch():
    mesh = plsc.VectorSubcoreMesh(core_axis_name="c", subcore_axis_name="s")

    @functools.partial(
        pl.kernel,
        out_type=(
            jax.ShapeDtypeStruct((P, D), jnp.float32),
            jax.ShapeDtypeStruct((P, WL), jnp.float32),
        ),
        mesh=mesh,
        scratch_types=[
            pltpu.VMEM((TOK, D), jnp.float32),
            pltpu.VMEM((TOK, WL), jnp.float32),
            pltpu.VMEM((TOK, WL), jnp.float32),
            pltpu.VMEM((TOK,), jnp.int32),
            pltpu.VMEM((TOK,), jnp.int32),
            pltpu.SemaphoreType.DMA,
        ],
    )
    def _sc_dispatch_k(z_hbm, dA_hbm, dB_hbm, wA_hbm, wB_hbm, xs_hbm, sw_hbm,
                       rows_v, wA_v, wB_v, idxA_v, idxB_v, sem):
        wid = lax.axis_index("s") * 2 + lax.axis_index("c")
        base = wid * TOK
        pltpu.sync_copy(z_hbm.at[pl.ds(base, TOK)], rows_v)
        pltpu.sync_copy(dA_hbm.at[pl.ds(base, TOK)], idxA_v)
        pltpu.sync_copy(dB_hbm.at[pl.ds(base, TOK)], idxB_v)
        pltpu.sync_copy(wA_hbm.at[pl.ds(base, TOK)], wA_v)
        pltpu.sync_copy(wB_hbm.at[pl.ds(base, TOK)], wB_v)
        c1 = pltpu.async_copy(rows_v, xs_hbm.at[idxA_v], sem)
        c2 = pltpu.async_copy(rows_v, xs_hbm.at[idxB_v], sem)
        c3 = pltpu.async_copy(wA_v, sw_hbm.at[idxA_v], sem)
        c4 = pltpu.async_copy(wB_v, sw_hbm.at[idxB_v], sem)
        c1.wait()
        c2.wait()
        c3.wait()
        c4.wait()

    return _sc_dispatch_k


def _sc_dispatch(z2, dA1, dB1, wA, wB):
    return _make_sc_dispatch()(z2, dA1, dB1, wA, wB)


def _grn_body(te_ref, tv_ref, x_ref, sw_ref, cc_ref, W2_ref, b2_ref, W3_ref,
              W1_ref, b1_ref, W4_ref, b4_ref, W5_ref, b5_ref, g_ref, bn_ref,
              y_ref):
    j = pl.program_id(0)

    @pl.when(tv_ref[j] == 1)
    def _():
        x = x_ref[...]
        cvec = jnp.dot(cc_ref[...], W3_ref[0],
                       preferred_element_type=jnp.float32) + b2_ref[0]
        h = jnp.dot(x, W2_ref[0], preferred_element_type=jnp.float32) + cvec
        h = jnp.where(h > 0, h, jnp.exp(jnp.minimum(h, 0.0)) - 1.0)
        h2 = jnp.dot(h, W1_ref[0],
                     preferred_element_type=jnp.float32) + b1_ref[0]
        a = jnp.dot(h2, W4_ref[0],
                    preferred_element_type=jnp.float32) + b4_ref[0]
        b = jnp.dot(h2, W5_ref[0],
                    preferred_element_type=jnp.float32) + b5_ref[0]
        glu = a * (1.0 / (1.0 + jnp.exp(-b)))
        r = x + glu
        mu = jnp.mean(r, axis=-1, keepdims=True)
        var = jnp.mean(r * r, axis=-1, keepdims=True) - mu * mu
        y = (r - mu) * lax.rsqrt(var + 1e-5) * g_ref[0] + bn_ref[0]
        y_ref[...] = y * sw_ref[:, 0:1]


def _run_grn(te, tv, xs, sw, c_c, W2, b2, W3, W1, b1, W4, b4, W5, b5,
             ln_g, ln_b):
    ee = lambda j, te_r, tv_r: (te_r[j], 0, 0)
    grid_spec = pltpu.PrefetchScalarGridSpec(
        num_scalar_prefetch=2,
        grid=(T,),
        in_specs=[
            pl.BlockSpec((MT, D), lambda j, te_r, tv_r: (j, 0)),
            pl.BlockSpec((MT, WL), lambda j, te_r, tv_r: (j, 0)),
            pl.BlockSpec((1, C), lambda j, te_r, tv_r: (0, 0)),
            pl.BlockSpec((1, D, H), ee),
            pl.BlockSpec((1, 1, H), ee),
            pl.BlockSpec((1, C, H), ee),
            pl.BlockSpec((1, H, D), ee),
            pl.BlockSpec((1, 1, D), ee),
            pl.BlockSpec((1, D, D), ee),
            pl.BlockSpec((1, 1, D), ee),
            pl.BlockSpec((1, D, D), ee),
            pl.BlockSpec((1, 1, D), ee),
            pl.BlockSpec((1, 1, D), ee),
            pl.BlockSpec((1, 1, D), ee),
        ],
        out_specs=pl.BlockSpec((MT, D), lambda j, te_r, tv_r: (j, 0)),
    )
    return pl.pallas_call(
        _grn_body,
        grid_spec=grid_spec,
        out_shape=jax.ShapeDtypeStruct((P, D), jnp.float32),
    )(te, tv, xs, sw, c_c, W2, b2.reshape(E, 1, H), W3, W1,
      b1.reshape(E, 1, D),
      W4, b4.reshape(E, 1, D), W5, b5.reshape(E, 1, D),
      ln_g.reshape(E, 1, D), ln_b.reshape(E, 1, D))


@functools.cache
def _make_sc_combine():
    mesh = plsc.VectorSubcoreMesh(core_axis_name="c", subcore_axis_name="s")

    @functools.partial(
        pl.kernel,
        out_type=jax.ShapeDtypeStruct((N, D), jnp.float32),
        mesh=mesh,
        scratch_types=[
            pltpu.VMEM((TOK, D), jnp.float32),
            pltpu.VMEM((TOK, D), jnp.float32),
            pltpu.VMEM((TOK,), jnp.int32),
            pltpu.VMEM((TOK,), jnp.int32),
            pltpu.SemaphoreType.DMA,
        ],
    )
    def _sc_combine_k(y_hbm, dA_hbm, dB_hbm, out_hbm,
                      bufA, bufB, idxA, idxB, sem):
        wid = lax.axis_index("s") * 2 + lax.axis_index("c")
        base = wid * TOK
        pltpu.sync_copy(dA_hbm.at[pl.ds(base, TOK)], idxA)
        pltpu.sync_copy(dB_hbm.at[pl.ds(base, TOK)], idxB)
        c1 = pltpu.async_copy(y_hbm.at[idxA], bufA, sem)
        c2 = pltpu.async_copy(y_hbm.at[idxB], bufB, sem)
        c1.wait()
        c2.wait()

        def row(r, carry):
            for k in range(D // L):
                sl = pl.ds(k * L, L)
                bufA[r, sl] = bufA[r, sl] + bufB[r, sl]
            return carry

        lax.fori_loop(0, TOK, row, 0)
        pltpu.sync_copy(bufA, out_hbm.at[pl.ds(base, TOK)])

    return _sc_combine_k


def _sc_combine(ys, dA1, dB1):
    return _make_sc_combine()(ys, dA1, dB1)


def kernel(z, c_c, router_W, router_b, W2, b2, W3, W1, b1, W4, b4, W5, b5,
           ln_g, ln_b):
    z2 = z.reshape(N, D)
    probs, dA, dB, wA, wB, te, tv = _run_plan(z2, router_W, router_b)
    dA1 = dA.reshape(N)
    dB1 = dB.reshape(N)
    xs, sw = _sc_dispatch(z2, dA1, dB1, wA, wB)
    ys = _run_grn(te.reshape(T), tv.reshape(T), xs, sw, c_c, W2, b2, W3, W1,
                  b1, W4, b4, W5, b5, ln_g, ln_b)
    out = _sc_combine(ys, dA1, dB1)
    return out.reshape(1, N, D), probs.reshape(1, N, E)


# speedup vs baseline: 3.4714x; 1.0268x over previous
import functools

import jax
import jax.numpy as jnp
from jax import lax
from jax.experimental import pallas as pl
from jax.experimental.pallas import tpu as pltpu
from jax.experimental.pallas import tpu_sc as plsc

N, D, C, H, E = 2048, 768, 32, 1536, 8
MT = 256
T = 24
P = MT * T
NW = 32
TOK = N // NW
L = 16
WL = 128


def _plan_body(z_ref, rw_ref, rb_ref, probs_ref, zp_ref, dA_ref, dB_ref,
               wA_ref, wB_ref, te_ref, tv_ref):
    zf = z_ref[...]
    rlo = zf[:, :D // 2].astype(jnp.bfloat16).astype(jnp.float32)
    rhi = zf[:, D // 2:].astype(jnp.bfloat16).astype(jnp.float32)
    ulo = lax.bitcast_convert_type(rlo, jnp.uint32) >> 16
    uhi = (lax.bitcast_convert_type(rhi, jnp.uint32)
           & jnp.uint32(0xFFFF0000))
    zp_ref[...] = lax.bitcast_convert_type(ulo | uhi, jnp.float32)
    logits = jnp.dot(z_ref[...], rw_ref[...],
                     preferred_element_type=jnp.float32) + rb_ref[...]
    m = jnp.max(logits, axis=-1, keepdims=True)
    ex = jnp.exp(logits - m)
    probs = ex / jnp.sum(ex, axis=-1, keepdims=True)
    probs_ref[...] = probs

    idx = lax.broadcasted_iota(jnp.int32, (N, E), 1)
    m1 = jnp.max(probs, axis=-1, keepdims=True)
    i1 = jnp.min(jnp.where(probs == m1, idx, E), axis=-1, keepdims=True)
    ohA = (idx == i1)
    pm = jnp.where(ohA, -1.0, probs)
    m2 = jnp.max(pm, axis=-1, keepdims=True)
    i2 = jnp.min(jnp.where(pm == m2, idx, E), axis=-1, keepdims=True)
    ohB = (idx == i2)
    s = m1 + m2 + 1e-9
    wA_ref[...] = jnp.broadcast_to(m1 / s, (N, WL))
    wB_ref[...] = jnp.broadcast_to(m2 / s, (N, WL))

    ohAf = ohA.astype(jnp.float32)
    ohBf = ohB.astype(jnp.float32)
    bs = 128
    li = (lax.broadcasted_iota(jnp.int32, (bs, bs), 0)
          > lax.broadcasted_iota(jnp.int32, (bs, bs), 1)).astype(jnp.float32)
    carry = jnp.zeros((1, E), jnp.float32)
    parts = {0: [], 1: []}
    for half, oh in ((0, ohAf), (1, ohBf)):
        for blk in range(N // bs):
            xb = oh[blk * bs:(blk + 1) * bs]
            excl = jnp.dot(li, xb, preferred_element_type=jnp.float32) + carry
            parts[half].append(jnp.sum(excl * xb, axis=1, keepdims=True))
            carry = carry + jnp.sum(xb, axis=0, keepdims=True)
    rankA = jnp.concatenate(parts[0], axis=0)
    rankB = jnp.concatenate(parts[1], axis=0)
    counts = carry.astype(jnp.int32)
    padded = ((counts + (MT - 1)) // MT) * MT
    ut8 = (lax.broadcasted_iota(jnp.int32, (E, E), 0)
           <= lax.broadcasted_iota(jnp.int32, (E, E), 1)).astype(jnp.float32)
    offs_incl = jnp.dot(padded.astype(jnp.float32), ut8,
                        preferred_element_type=jnp.float32).astype(jnp.int32)
    offs_excl = offs_incl - padded
    offs_excl_f = offs_excl.astype(jnp.float32)
    dA_ref[...] = (jnp.sum(ohAf * offs_excl_f, axis=1, keepdims=True)
                   + rankA).astype(jnp.int32)
    dB_ref[...] = (jnp.sum(ohBf * offs_excl_f, axis=1, keepdims=True)
                   + rankB).astype(jnp.int32)

    jv = lax.broadcasted_iota(jnp.int32, (1, T), 1)
    ot = offs_excl // MT
    acc = jnp.full((1, T), -1, jnp.int32)
    for e in range(E):
        acc = acc + (jv >= ot[0:1, e:e + 1]).astype(jnp.int32)
    te_ref[...] = acc
    tv_ref[...] = (jv < offs_incl[0:1, E - 1:E] // MT).astype(jnp.int32)


def _run_plan(z2, router_W, router_b):
    return pl.pallas_call(
        _plan_body,
        out_shape=(
            jax.ShapeDtypeStruct((N, E), jnp.float32),
            jax.ShapeDtypeStruct((N, D // 2), jnp.float32),
            jax.ShapeDtypeStruct((N, 1), jnp.int32),
            jax.ShapeDtypeStruct((N, 1), jnp.int32),
            jax.ShapeDtypeStruct((N, WL), jnp.float32),
            jax.ShapeDtypeStruct((N, WL), jnp.float32),
            jax.ShapeDtypeStruct((1, T), jnp.int32),
            jax.ShapeDtypeStruct((1, T), jnp.int32),
        ),
    )(z2, router_W, router_b.reshape(1, E))


@functools.cache
def _make_sc_dispatch():
    mesh = plsc.VectorSubcoreMesh(core_axis_name="c", subcore_axis_name="s")

    @functools.partial(
        pl.kernel,
        out_type=(
            jax.ShapeDtypeStruct((P, D // 2), jnp.float32),
            jax.ShapeDtypeStruct((P, WL), jnp.float32),
        ),
        mesh=mesh,
        scratch_types=[
            pltpu.VMEM((TOK, D // 2), jnp.float32),
            pltpu.VMEM((TOK, WL), jnp.float32),
            pltpu.VMEM((TOK, WL), jnp.float32),
            pltpu.VMEM((TOK,), jnp.int32),
            pltpu.VMEM((TOK,), jnp.int32),
            pltpu.SemaphoreType.DMA,
        ],
    )
    def _sc_dispatch_k(z_hbm, dA_hbm, dB_hbm, wA_hbm, wB_hbm, xs_hbm, sw_hbm,
                       rows_v, wA_v, wB_v, idxA_v, idxB_v, sem):
        wid = lax.axis_index("s") * 2 + lax.axis_index("c")
        base = wid * TOK
        pltpu.sync_copy(z_hbm.at[pl.ds(base, TOK)], rows_v)
        pltpu.sync_copy(dA_hbm.at[pl.ds(base, TOK)], idxA_v)
        pltpu.sync_copy(dB_hbm.at[pl.ds(base, TOK)], idxB_v)
        pltpu.sync_copy(wA_hbm.at[pl.ds(base, TOK)], wA_v)
        pltpu.sync_copy(wB_hbm.at[pl.ds(base, TOK)], wB_v)
        c1 = pltpu.async_copy(rows_v, xs_hbm.at[idxA_v], sem)
        c2 = pltpu.async_copy(rows_v, xs_hbm.at[idxB_v], sem)
        c3 = pltpu.async_copy(wA_v, sw_hbm.at[idxA_v], sem)
        c4 = pltpu.async_copy(wB_v, sw_hbm.at[idxB_v], sem)
        c1.wait()
        c2.wait()
        c3.wait()
        c4.wait()

    return _sc_dispatch_k


def _sc_dispatch(z2, dA1, dB1, wA, wB):
    return _make_sc_dispatch()(z2, dA1, dB1, wA, wB)


def _grn_body(te_ref, tv_ref, x_ref, sw_ref, cc_ref, W2_ref, b2_ref, W3_ref,
              W1_ref, b1_ref, W4_ref, b4_ref, W5_ref, b5_ref, g_ref, bn_ref,
              y_ref):
    j = pl.program_id(0)

    @pl.when(tv_ref[j] == 1)
    def _():
        u = lax.bitcast_convert_type(x_ref[...], jnp.uint32)
        xlo = lax.bitcast_convert_type(u << 16, jnp.float32)
        xhi = lax.bitcast_convert_type(u & jnp.uint32(0xFFFF0000),
                                       jnp.float32)
        x = jnp.concatenate([xlo, xhi], axis=1)
        cvec = jnp.dot(cc_ref[...], W3_ref[0],
                       preferred_element_type=jnp.float32) + b2_ref[0]
        h = jnp.dot(x, W2_ref[0], preferred_element_type=jnp.float32) + cvec
        h = jnp.where(h > 0, h, jnp.exp(jnp.minimum(h, 0.0)) - 1.0)
        h2 = jnp.dot(h, W1_ref[0],
                     preferred_element_type=jnp.float32) + b1_ref[0]
        a = jnp.dot(h2, W4_ref[0],
                    preferred_element_type=jnp.float32) + b4_ref[0]
        b = jnp.dot(h2, W5_ref[0],
                    preferred_element_type=jnp.float32) + b5_ref[0]
        glu = a * (1.0 / (1.0 + jnp.exp(-b)))
        r = x + glu
        mu = jnp.mean(r, axis=-1, keepdims=True)
        var = jnp.mean(r * r, axis=-1, keepdims=True) - mu * mu
        y = (r - mu) * lax.rsqrt(var + 1e-5) * g_ref[0] + bn_ref[0]
        y_ref[...] = y * sw_ref[:, 0:1]


def _run_grn(te, tv, xs, sw, c_c, W2, b2, W3, W1, b1, W4, b4, W5, b5,
             ln_g, ln_b):
    ee = lambda j, te_r, tv_r: (te_r[j], 0, 0)
    grid_spec = pltpu.PrefetchScalarGridSpec(
        num_scalar_prefetch=2,
        grid=(T,),
        in_specs=[
            pl.BlockSpec((MT, D // 2),
                         lambda j, te_r, tv_r: (jnp.where(tv_r[j] == 1, j, T - 1), 0)),
            pl.BlockSpec((MT, WL),
                         lambda j, te_r, tv_r: (jnp.where(tv_r[j] == 1, j, T - 1), 0)),
            pl.BlockSpec((1, C), lambda j, te_r, tv_r: (0, 0)),
            pl.BlockSpec((1, D, H), ee),
            pl.BlockSpec((1, 1, H), ee),
            pl.BlockSpec((1, C, H), ee),
            pl.BlockSpec((1, H, D), ee),
            pl.BlockSpec((1, 1, D), ee),
            pl.BlockSpec((1, D, D), ee),
            pl.BlockSpec((1, 1, D), ee),
            pl.BlockSpec((1, D, D), ee),
            pl.BlockSpec((1, 1, D), ee),
            pl.BlockSpec((1, 1, D), ee),
            pl.BlockSpec((1, 1, D), ee),
        ],
        out_specs=pl.BlockSpec(
            (MT, D), lambda j, te_r, tv_r: (jnp.where(tv_r[j] == 1, j, T - 1), 0)),
    )
    return pl.pallas_call(
        _grn_body,
        grid_spec=grid_spec,
        out_shape=jax.ShapeDtypeStruct((P, D), jnp.float32),
    )(te, tv, xs, sw, c_c, W2, b2.reshape(E, 1, H), W3, W1,
      b1.reshape(E, 1, D),
      W4, b4.reshape(E, 1, D), W5, b5.reshape(E, 1, D),
      ln_g.reshape(E, 1, D), ln_b.reshape(E, 1, D))


@functools.cache
def _make_sc_combine():
    mesh = plsc.VectorSubcoreMesh(core_axis_name="c", subcore_axis_name="s")

    @functools.partial(
        pl.kernel,
        out_type=jax.ShapeDtypeStruct((N, D), jnp.float32),
        mesh=mesh,
        scratch_types=[
            pltpu.VMEM((TOK, D), jnp.float32),
            pltpu.VMEM((TOK, D), jnp.float32),
            pltpu.VMEM((TOK,), jnp.int32),
            pltpu.VMEM((TOK,), jnp.int32),
            pltpu.SemaphoreType.DMA,
        ],
    )
    def _sc_combine_k(y_hbm, dA_hbm, dB_hbm, out_hbm,
                      bufA, bufB, idxA, idxB, sem):
        wid = lax.axis_index("s") * 2 + lax.axis_index("c")
        base = wid * TOK
        pltpu.sync_copy(dA_hbm.at[pl.ds(base, TOK)], idxA)
        pltpu.sync_copy(dB_hbm.at[pl.ds(base, TOK)], idxB)
        c1 = pltpu.async_copy(y_hbm.at[idxA], bufA, sem)
        c2 = pltpu.async_copy(y_hbm.at[idxB], bufB, sem)
        c1.wait()
        c2.wait()

        def row(r, carry):
            for k in range(D // L):
                sl = pl.ds(k * L, L)
                bufA[r, sl] = bufA[r, sl] + bufB[r, sl]
            return carry

        lax.fori_loop(0, TOK, row, 0)
        pltpu.sync_copy(bufA, out_hbm.at[pl.ds(base, TOK)])

    return _sc_combine_k


def _sc_combine(ys, dA1, dB1):
    return _make_sc_combine()(ys, dA1, dB1)


def kernel(z, c_c, router_W, router_b, W2, b2, W3, W1, b1, W4, b4, W5, b5,
           ln_g, ln_b):
    z2 = z.reshape(N, D)
    probs, zp, dA, dB, wA, wB, te, tv = _run_plan(z2, router_W, router_b)
    dA1 = dA.reshape(N)
    dB1 = dB.reshape(N)
    xs, sw = _sc_dispatch(zp, dA1, dB1, wA, wB)
    ys = _run_grn(te.reshape(T), tv.reshape(T), xs, sw, c_c, W2, b2, W3, W1,
                  b1, W4, b4, W5, b5, ln_g, ln_b)
    out = _sc_combine(ys, dA1, dB1)
    return out.reshape(1, N, D), probs.reshape(1, N, E)


# speedup vs baseline: 3.5077x; 1.0105x over previous
import functools

import jax
import jax.numpy as jnp
from jax import lax
from jax.experimental import pallas as pl
from jax.experimental.pallas import tpu as pltpu
from jax.experimental.pallas import tpu_sc as plsc

N, D, C, H, E = 2048, 768, 32, 1536, 8
MT = 256
T = 24
P = MT * T
NW = 32
TOK = N // NW
L = 16
WL = 128


def _plan_body(z_ref, rw_ref, rb_ref, probs_ref, zp_ref, dA_ref, dB_ref,
               wA_ref, wB_ref, te_ref, tv_ref):
    zf = z_ref[0]
    rlo = zf[:, :D // 2].astype(jnp.bfloat16).astype(jnp.float32)
    rhi = zf[:, D // 2:].astype(jnp.bfloat16).astype(jnp.float32)
    ulo = lax.bitcast_convert_type(rlo, jnp.uint32) >> 16
    uhi = (lax.bitcast_convert_type(rhi, jnp.uint32)
           & jnp.uint32(0xFFFF0000))
    zp_ref[...] = lax.bitcast_convert_type(ulo | uhi, jnp.float32)
    logits = jnp.dot(zf, rw_ref[...],
                     preferred_element_type=jnp.float32) + rb_ref[...]
    m = jnp.max(logits, axis=-1, keepdims=True)
    ex = jnp.exp(logits - m)
    probs = ex / jnp.sum(ex, axis=-1, keepdims=True)
    probs_ref[...] = probs

    idx = lax.broadcasted_iota(jnp.int32, (N, E), 1)
    m1 = jnp.max(probs, axis=-1, keepdims=True)
    i1 = jnp.min(jnp.where(probs == m1, idx, E), axis=-1, keepdims=True)
    ohA = (idx == i1)
    pm = jnp.where(ohA, -1.0, probs)
    m2 = jnp.max(pm, axis=-1, keepdims=True)
    i2 = jnp.min(jnp.where(pm == m2, idx, E), axis=-1, keepdims=True)
    ohB = (idx == i2)
    s = m1 + m2 + 1e-9
    wA_ref[...] = jnp.broadcast_to(m1 / s, (N, WL))
    wB_ref[...] = jnp.broadcast_to(m2 / s, (N, WL))

    ohAf = ohA.astype(jnp.float32)
    ohBf = ohB.astype(jnp.float32)
    bs = 128
    li = (lax.broadcasted_iota(jnp.int32, (bs, bs), 0)
          > lax.broadcasted_iota(jnp.int32, (bs, bs), 1)).astype(jnp.float32)
    carry = jnp.zeros((1, E), jnp.float32)
    parts = {0: [], 1: []}
    for half, oh in ((0, ohAf), (1, ohBf)):
        for blk in range(N // bs):
            xb = oh[blk * bs:(blk + 1) * bs]
            excl = jnp.dot(li, xb, preferred_element_type=jnp.float32) + carry
            parts[half].append(jnp.sum(excl * xb, axis=1, keepdims=True))
            carry = carry + jnp.sum(xb, axis=0, keepdims=True)
    rankA = jnp.concatenate(parts[0], axis=0)
    rankB = jnp.concatenate(parts[1], axis=0)
    counts = carry.astype(jnp.int32)
    padded = ((counts + (MT - 1)) // MT) * MT
    ut8 = (lax.broadcasted_iota(jnp.int32, (E, E), 0)
           <= lax.broadcasted_iota(jnp.int32, (E, E), 1)).astype(jnp.float32)
    offs_incl = jnp.dot(padded.astype(jnp.float32), ut8,
                        preferred_element_type=jnp.float32).astype(jnp.int32)
    offs_excl = offs_incl - padded
    offs_excl_f = offs_excl.astype(jnp.float32)
    dA_ref[...] = (jnp.sum(ohAf * offs_excl_f, axis=1, keepdims=True)
                   + rankA).astype(jnp.int32)
    dB_ref[...] = (jnp.sum(ohBf * offs_excl_f, axis=1, keepdims=True)
                   + rankB).astype(jnp.int32)

    jv = lax.broadcasted_iota(jnp.int32, (1, T), 1)
    ot = offs_excl // MT
    acc = jnp.full((1, T), -1, jnp.int32)
    for e in range(E):
        acc = acc + (jv >= ot[0:1, e:e + 1]).astype(jnp.int32)
    te_ref[...] = acc
    tv_ref[...] = (jv < offs_incl[0:1, E - 1:E] // MT).astype(jnp.int32)


def _run_plan(z3, router_W, router_b):
    return pl.pallas_call(
        _plan_body,
        out_shape=(
            jax.ShapeDtypeStruct((N, E), jnp.float32),
            jax.ShapeDtypeStruct((N, D // 2), jnp.float32),
            jax.ShapeDtypeStruct((N, 1), jnp.int32),
            jax.ShapeDtypeStruct((N, 1), jnp.int32),
            jax.ShapeDtypeStruct((N, WL), jnp.float32),
            jax.ShapeDtypeStruct((N, WL), jnp.float32),
            jax.ShapeDtypeStruct((1, T), jnp.int32),
            jax.ShapeDtypeStruct((1, T), jnp.int32),
        ),
    )(z3, router_W, router_b.reshape(1, E))


@functools.cache
def _make_sc_dispatch():
    mesh = plsc.VectorSubcoreMesh(core_axis_name="c", subcore_axis_name="s")

    @functools.partial(
        pl.kernel,
        out_type=(
            jax.ShapeDtypeStruct((P, D // 2), jnp.float32),
            jax.ShapeDtypeStruct((P, WL), jnp.float32),
        ),
        mesh=mesh,
        scratch_types=[
            pltpu.VMEM((TOK, D // 2), jnp.float32),
            pltpu.VMEM((TOK, WL), jnp.float32),
            pltpu.VMEM((TOK, WL), jnp.float32),
            pltpu.VMEM((TOK,), jnp.int32),
            pltpu.VMEM((TOK,), jnp.int32),
            pltpu.SemaphoreType.DMA,
        ],
    )
    def _sc_dispatch_k(z_hbm, dA_hbm, dB_hbm, wA_hbm, wB_hbm, xs_hbm, sw_hbm,
                       rows_v, wA_v, wB_v, idxA_v, idxB_v, sem):
        wid = lax.axis_index("s") * 2 + lax.axis_index("c")
        base = wid * TOK
        pltpu.sync_copy(z_hbm.at[pl.ds(base, TOK)], rows_v)
        pltpu.sync_copy(dA_hbm.at[pl.ds(base, TOK)], idxA_v)
        pltpu.sync_copy(dB_hbm.at[pl.ds(base, TOK)], idxB_v)
        pltpu.sync_copy(wA_hbm.at[pl.ds(base, TOK)], wA_v)
        pltpu.sync_copy(wB_hbm.at[pl.ds(base, TOK)], wB_v)
        c1 = pltpu.async_copy(rows_v, xs_hbm.at[idxA_v], sem)
        c2 = pltpu.async_copy(rows_v, xs_hbm.at[idxB_v], sem)
        c3 = pltpu.async_copy(wA_v, sw_hbm.at[idxA_v], sem)
        c4 = pltpu.async_copy(wB_v, sw_hbm.at[idxB_v], sem)
        c1.wait()
        c2.wait()
        c3.wait()
        c4.wait()

    return _sc_dispatch_k


def _sc_dispatch(z2, dA1, dB1, wA, wB):
    return _make_sc_dispatch()(z2, dA1, dB1, wA, wB)


def _grn_body(te_ref, tv_ref, x_ref, sw_ref, cc_ref, W2_ref, b2_ref, W3_ref,
              W1_ref, b1_ref, W4_ref, b4_ref, W5_ref, b5_ref, g_ref, bn_ref,
              y_ref):
    j = pl.program_id(0)

    @pl.when(tv_ref[0, j] == 1)
    def _():
        u = lax.bitcast_convert_type(x_ref[...], jnp.uint32)
        xlo = lax.bitcast_convert_type(u << 16, jnp.float32)
        xhi = lax.bitcast_convert_type(u & jnp.uint32(0xFFFF0000),
                                       jnp.float32)
        x = jnp.concatenate([xlo, xhi], axis=1)
        cvec = jnp.dot(cc_ref[...], W3_ref[0],
                       preferred_element_type=jnp.float32) + b2_ref[0]
        h = jnp.dot(x, W2_ref[0], preferred_element_type=jnp.float32) + cvec
        h = jnp.where(h > 0, h, jnp.exp(jnp.minimum(h, 0.0)) - 1.0)
        h2 = jnp.dot(h, W1_ref[0],
                     preferred_element_type=jnp.float32) + b1_ref[0]
        a = jnp.dot(h2, W4_ref[0],
                    preferred_element_type=jnp.float32) + b4_ref[0]
        b = jnp.dot(h2, W5_ref[0],
                    preferred_element_type=jnp.float32) + b5_ref[0]
        glu = a * (1.0 / (1.0 + jnp.exp(-b)))
        r = x + glu
        mu = jnp.mean(r, axis=-1, keepdims=True)
        var = jnp.mean(r * r, axis=-1, keepdims=True) - mu * mu
        y = (r - mu) * lax.rsqrt(var + 1e-5) * g_ref[0] + bn_ref[0]
        y_ref[...] = y * sw_ref[:, 0:1]


def _run_grn(te, tv, xs, sw, c_c, W2, b2, W3, W1, b1, W4, b4, W5, b5,
             ln_g, ln_b):
    ee = lambda j, te_r, tv_r: (te_r[0, j], 0, 0)
    grid_spec = pltpu.PrefetchScalarGridSpec(
        num_scalar_prefetch=2,
        grid=(T,),
        in_specs=[
            pl.BlockSpec((MT, D // 2),
                         lambda j, te_r, tv_r: (jnp.where(tv_r[0, j] == 1, j, T - 1), 0)),
            pl.BlockSpec((MT, WL),
                         lambda j, te_r, tv_r: (jnp.where(tv_r[0, j] == 1, j, T - 1), 0)),
            pl.BlockSpec((1, C), lambda j, te_r, tv_r: (0, 0)),
            pl.BlockSpec((1, D, H), ee),
            pl.BlockSpec((1, 1, H), ee),
            pl.BlockSpec((1, C, H), ee),
            pl.BlockSpec((1, H, D), ee),
            pl.BlockSpec((1, 1, D), ee),
            pl.BlockSpec((1, D, D), ee),
            pl.BlockSpec((1, 1, D), ee),
            pl.BlockSpec((1, D, D), ee),
            pl.BlockSpec((1, 1, D), ee),
            pl.BlockSpec((1, 1, D), ee),
            pl.BlockSpec((1, 1, D), ee),
        ],
        out_specs=pl.BlockSpec(
            (MT, D), lambda j, te_r, tv_r: (jnp.where(tv_r[0, j] == 1, j, T - 1), 0)),
    )
    return pl.pallas_call(
        _grn_body,
        grid_spec=grid_spec,
        out_shape=jax.ShapeDtypeStruct((P, D), jnp.float32),
    )(te, tv, xs, sw, c_c, W2, b2.reshape(E, 1, H), W3, W1,
      b1.reshape(E, 1, D),
      W4, b4.reshape(E, 1, D), W5, b5.reshape(E, 1, D),
      ln_g.reshape(E, 1, D), ln_b.reshape(E, 1, D))


@functools.cache
def _make_sc_combine():
    mesh = plsc.VectorSubcoreMesh(core_axis_name="c", subcore_axis_name="s")

    @functools.partial(
        pl.kernel,
        out_type=jax.ShapeDtypeStruct((N, D), jnp.float32),
        mesh=mesh,
        scratch_types=[
            pltpu.VMEM((TOK, D), jnp.float32),
            pltpu.VMEM((TOK, D), jnp.float32),
            pltpu.VMEM((TOK,), jnp.int32),
            pltpu.VMEM((TOK,), jnp.int32),
            pltpu.SemaphoreType.DMA,
        ],
    )
    def _sc_combine_k(y_hbm, dA_hbm, dB_hbm, out_hbm,
                      bufA, bufB, idxA, idxB, sem):
        wid = lax.axis_index("s") * 2 + lax.axis_index("c")
        base = wid * TOK
        pltpu.sync_copy(dA_hbm.at[pl.ds(base, TOK)], idxA)
        pltpu.sync_copy(dB_hbm.at[pl.ds(base, TOK)], idxB)
        c1 = pltpu.async_copy(y_hbm.at[idxA], bufA, sem)
        c2 = pltpu.async_copy(y_hbm.at[idxB], bufB, sem)
        c1.wait()
        c2.wait()

        def row(r, carry):
            for k in range(D // L):
                sl = pl.ds(k * L, L)
                bufA[r, sl] = bufA[r, sl] + bufB[r, sl]
            return carry

        lax.fori_loop(0, TOK, row, 0)
        pltpu.sync_copy(bufA, out_hbm.at[pl.ds(base, TOK)])

    return _sc_combine_k


def _sc_combine(ys, dA1, dB1):
    return _make_sc_combine()(ys, dA1, dB1)


def kernel(z, c_c, router_W, router_b, W2, b2, W3, W1, b1, W4, b4, W5, b5,
           ln_g, ln_b):
    probs, zp, dA, dB, wA, wB, te, tv = _run_plan(z, router_W, router_b)
    dA1 = dA.reshape(N)
    dB1 = dB.reshape(N)
    xs, sw = _sc_dispatch(zp, dA1, dB1, wA, wB)
    ys = _run_grn(te, tv, xs, sw, c_c, W2, b2, W3, W1,
                  b1, W4, b4, W5, b5, ln_g, ln_b)
    out = _sc_combine(ys, dA1, dB1)
    return out.reshape(1, N, D), probs.reshape(1, N, E)
